# baseline (speedup 1.0000x reference)
import functools

import jax
import jax.numpy as jnp
import numpy as np
from jax import lax
from jax.experimental import pallas as pl
from jax.experimental.pallas import tpu as pltpu
from jax.experimental.pallas import tpu_sc as plsc

N = 4096
K = 128
R2 = 64.0
NW = 32
TPW = N // NW
CAP = N + 32
BIG = np.float32(3.0e38)
BIGI = np.int32(2**30)


def _sc_radius_graph(posx, posy, posz, batch, bounds):
    mesh = plsc.VectorSubcoreMesh(core_axis_name="c", subcore_axis_name="s")

    @functools.partial(
        pl.kernel,
        out_type=(
            jax.ShapeDtypeStruct((2 * N * K,), jnp.int32),
            jax.ShapeDtypeStruct((N * K,), jnp.float32),
        ),
        mesh=mesh,
        compiler_params=pltpu.CompilerParams(needs_layout_passes=False),
        scratch_types=(
            pltpu.VMEM((N,), jnp.float32),
            pltpu.VMEM((N,), jnp.float32),
            pltpu.VMEM((N,), jnp.float32),
            pltpu.VMEM((TPW,), jnp.int32),
            pltpu.VMEM((16,), jnp.int32),
            pltpu.VMEM((TPW,), jnp.int32),
            pltpu.VMEM((TPW,), jnp.int32),
            pltpu.VMEM((CAP,), jnp.float32),
            pltpu.VMEM((CAP,), jnp.int32),
            pltpu.VMEM((CAP,), jnp.float32),
            pltpu.VMEM((CAP,), jnp.int32),
            pltpu.VMEM((CAP,), jnp.float32),
            pltpu.VMEM((CAP,), jnp.int32),
            pltpu.VMEM((CAP,), jnp.float32),
            pltpu.VMEM((CAP,), jnp.int32),
            pltpu.VMEM((CAP,), jnp.float32),
            pltpu.VMEM((CAP,), jnp.int32),
            pltpu.VMEM((CAP,), jnp.float32),
            pltpu.VMEM((CAP,), jnp.int32),
            pltpu.VMEM((CAP,), jnp.float32),
            pltpu.VMEM((CAP,), jnp.int32),
            pltpu.VMEM((CAP,), jnp.float32),
            pltpu.VMEM((CAP,), jnp.int32),
            pltpu.VMEM((TPW * K,), jnp.int32),
            pltpu.VMEM((TPW * K,), jnp.int32),
            pltpu.VMEM((TPW * K,), jnp.float32),
        ),
    )
    def body(posx_h, posy_h, posz_h, batch_h, bounds_h, out_ei, out_dist,
             px, py, pz, bt, bv, ss, se, cd2, cidx, cd2b, cidxb, cd2c,
             cidxc, cd2d, cidxd, cd2e, cidxe, cd2f, cidxf, cd2g, cidxg,
             cd2h, cidxh, src_loc, tgt_loc, dist_loc):
        wid = lax.axis_index("s") * 2 + lax.axis_index("c")
        base = wid * TPW
        pltpu.sync_copy(posx_h, px)
        pltpu.sync_copy(posy_h, py)
        pltpu.sync_copy(posz_h, pz)
        pltpu.sync_copy(batch_h.at[pl.ds(base, TPW)], bt)
        pltpu.sync_copy(bounds_h, bv)

        neg1 = jnp.full((16,), -1, jnp.int32)
        zeros = jnp.zeros((16,), jnp.float32)
        one = jnp.full((16,), 1, jnp.int32)

        def bounds_body(k, carry):
            b = bt[pl.ds(k * 16, 16)]
            ss[pl.ds(k * 16, 16)] = plsc.load_gather(bv, [b])
            se[pl.ds(k * 16, 16)] = plsc.load_gather(bv, [b + one])
            return carry

        lax.fori_loop(0, TPW // 16, bounds_body, 0)

        def init_body(k, carry):
            src_loc[pl.ds(k * 16, 16)] = neg1
            tgt_loc[pl.ds(k * 16, 16)] = neg1
            dist_loc[pl.ds(k * 16, 16)] = zeros
            return carry

        lax.fori_loop(0, TPW * K // 16, init_body, 0)

        iota = lax.iota(jnp.int32, 16)
        lane0 = iota == 0

        def splat(x, dtype=jnp.int32):
            return jnp.full((16,), x, dtype)

        def post_process(cnt, bd2, bidx, t, i_s):
            bd2[pl.ds(cnt, 16)] = jnp.full((16,), BIG, jnp.float32)
            row = t * K

            def slow_select():
                nsel = jnp.minimum(cnt, K)
                nch = (cnt + 15) >> 4

                def sel_body(sl, carry):
                    def minpass(c, macc):
                        return jnp.minimum(macc, bd2[pl.ds(c * 16, 16)])

                    macc = lax.fori_loop(0, nch, minpass,
                                         jnp.full((16,), BIG, jnp.float32))
                    m = jnp.min(macc)

                    def idxpass(c, jacc):
                        d2c = bd2[pl.ds(c * 16, 16)]
                        ic = bidx[pl.ds(c * 16, 16)]
                        return jnp.minimum(jacc, jnp.where(d2c == m, ic, BIGI))

                    jacc = lax.fori_loop(0, nch, idxpass,
                                         jnp.full((16,), BIGI, jnp.int32))
                    jmin = jnp.min(jacc)
                    p_s = splat(row + sl)
                    plsc.store_scatter(src_loc, [p_s], splat(jmin), mask=lane0)
                    plsc.store_scatter(dist_loc, [p_s], splat(m, jnp.float32),
                                       mask=lane0)

                    def knock(c, kc):
                        d2c = bd2[pl.ds(c * 16, 16)]
                        ic = bidx[pl.ds(c * 16, 16)]
                        hit = (d2c == m) & (ic == jmin)
                        bd2[pl.ds(c * 16, 16)] = jnp.where(hit, BIG, d2c)
                        return kc

                    lax.fori_loop(0, nch, knock, 0)
                    return carry

                lax.fori_loop(0, nsel, sel_body, 0)

            def tie_fallback(tie):
                anytie = plsc.all_reduce_population_count(tie)[0] > 0

                @pl.when(anytie)
                def _():
                    slow_select()

            @pl.when(cnt <= 16)
            def _():
                sk, sv = plsc.sort_key_val(bd2[pl.ds(0, 16)],
                                           bidx[pl.ds(0, 16)])
                pref = iota < cnt
                plsc.store_compressed(src_loc.at[pl.ds(row, 16)], sv,
                                      mask=pref)
                plsc.store_compressed(dist_loc.at[pl.ds(row, 16)], sk,
                                      mask=pref)
                a = dist_loc[pl.ds(row, 16)]
                b = dist_loc[pl.ds(row + 1, 16)]
                tie_fallback((a == b) & (iota < cnt - 1))

            @pl.when((cnt > 16) & (cnt <= 32))
            def _():
                ad, ai = plsc.sort_key_val(bd2[pl.ds(0, 16)],
                                           bidx[pl.ds(0, 16)])
                bd, bi = plsc.sort_key_val(bd2[pl.ds(16, 16)],
                                           bidx[pl.ds(16, 16)])
                brd = jnp.flip(bd)
                bri = jnp.flip(bi)
                sel = (ad < brd) | ((ad == brd) & (ai < bri))
                ld = jnp.where(sel, ad, brd)
                li = jnp.where(sel, ai, bri)
                hd = jnp.where(sel, brd, ad)
                hi = jnp.where(sel, bri, ai)
                ld, li = plsc.sort_key_val(ld, li)
                hd, hi = plsc.sort_key_val(hd, hi)
                src_loc[pl.ds(row, 16)] = li
                dist_loc[pl.ds(row, 16)] = ld
                pref = iota < cnt - 16
                plsc.store_compressed(src_loc.at[pl.ds(row + 16, 16)], hi,
                                      mask=pref)
                plsc.store_compressed(dist_loc.at[pl.ds(row + 16, 16)], hd,
                                      mask=pref)
                a1 = dist_loc[pl.ds(row, 16)]
                b1 = dist_loc[pl.ds(row + 1, 16)]
                a2 = dist_loc[pl.ds(row + 16, 16)]
                b2 = dist_loc[pl.ds(row + 17, 16)]
                tie1 = (a1 == b1) & (iota < 15)
                tie2 = (a2 == b2) & (iota < cnt - 17)
                tie_fallback(tie1 | tie2)

            @pl.when(cnt > 32)
            def _():
                slow_select()

            def sqrt_body(k, carry2):
                x = dist_loc[pl.ds(row + k * 16, 16)]
                sv = src_loc[pl.ds(row + k * 16, 16)]
                xm = jnp.maximum(x, jnp.float32(1e-12))
                ib = plsc.bitcast(xm, jnp.int32)
                r = plsc.bitcast(jnp.int32(0x5F3759DF) - (ib >> 1),
                                 jnp.float32)
                hx = xm * jnp.float32(0.5)
                r = r * (jnp.float32(1.5) - hx * r * r)
                r = r * (jnp.float32(1.5) - hx * r * r)
                r = r * (jnp.float32(1.5) - hx * r * r)
                d = xm * r
                d = jnp.float32(0.5) * (d + xm / d)
                sel = sv >= 0
                dist_loc[pl.ds(row + k * 16, 16)] = jnp.where(
                    sel, d, jnp.float32(0.0))
                tgt_loc[pl.ds(row + k * 16, 16)] = jnp.where(sel, i_s, neg1)
                return carry2

            lax.fori_loop(0, (jnp.minimum(cnt, K) + 15) >> 4, sqrt_body, 0)

        bufs = ((cd2, cidx), (cd2b, cidxb), (cd2c, cidxc), (cd2d, cidxd),
                (cd2e, cidxe), (cd2f, cidxf), (cd2g, cidxg), (cd2h, cidxh))
        NT = len(bufs)

        def group_body(tg, carry):
            ts = [NT * tg + u for u in range(NT)]
            i_ss = [splat(base + t) for t in ts]
            xs = [plsc.load_gather(px, [i_s]) for i_s in i_ss]
            ys = [plsc.load_gather(py, [i_s]) for i_s in i_ss]
            zs = [plsc.load_gather(pz, [i_s]) for i_s in i_ss]
            s0s = [plsc.load_gather(ss, [splat(t)])[0] for t in ts]
            s1s = [plsc.load_gather(se, [splat(t)])[0] for t in ts]
            lo = s0s[0]
            hi = s1s[0]
            for u in range(1, NT):
                lo = jnp.minimum(lo, s0s[u])
                hi = jnp.maximum(hi, s1s[u])
            c0 = lo & jnp.int32(-16)
            nsteps = (hi - c0 + 15) >> 4

            def make_chunk_body(shared_bounds):
                def chunk_body(c, cnts):
                    cb = c0 + c * 16
                    jv = cb + iota
                    pxc = px[pl.ds(cb, 16)]
                    pyc = py[pl.ds(cb, 16)]
                    pzc = pz[pl.ds(cb, 16)]
                    if shared_bounds:
                        sm = (jv >= lo) & (jv < hi)
                    out = []
                    for u in range(NT):
                        dx = pxc - xs[u]
                        dy = pyc - ys[u]
                        dz = pzc - zs[u]
                        d2 = (dx * dx + dy * dy) + dz * dz
                        v = (d2 <= R2) & (jv != base + ts[u])
                        if shared_bounds:
                            v &= sm
                        else:
                            v &= (jv >= s0s[u]) & (jv < s1s[u])
                        nv = plsc.all_reduce_population_count(v)[0]
                        bd2, bidx = bufs[u]
                        plsc.store_compressed(bd2.at[pl.ds(cnts[u], 16)], d2,
                                              mask=v)
                        plsc.store_compressed(bidx.at[pl.ds(cnts[u], 16)], jv,
                                              mask=v)
                        out.append(cnts[u] + nv)
                    return tuple(out)

                return chunk_body

            zero_cnts = tuple(jnp.int32(0) for _ in range(NT))
            cnts = lax.fori_loop(0, nsteps, make_chunk_body(False), zero_cnts)
            for u in range(NT):
                post_process(cnts[u], bufs[u][0], bufs[u][1], ts[u], i_ss[u])
            return carry

        lax.fori_loop(0, TPW // NT, group_body, 0)

        pltpu.sync_copy(src_loc, out_ei.at[pl.ds(base * K, TPW * K)])
        pltpu.sync_copy(tgt_loc, out_ei.at[pl.ds(N * K + base * K, TPW * K)])
        pltpu.sync_copy(dist_loc, out_dist.at[pl.ds(base * K, TPW * K)])

    return body(posx, posy, posz, batch, bounds)


def kernel(pos, batch):
    batch = batch.astype(jnp.int32)
    posx = pos[:, 0]
    posy = pos[:, 1]
    posz = pos[:, 2]
    bounds = jnp.sum(
        (batch[None, :] < jnp.arange(9, dtype=jnp.int32)[:, None]).astype(jnp.int32),
        axis=1)
    bounds16 = jnp.pad(bounds, (0, 7))
    ei, dist = _sc_radius_graph(posx, posy, posz, batch, bounds16)
    return ei.reshape(2, N * K), dist.reshape(N, K)


# speedup vs baseline: 1.3825x; 1.3825x over previous
import functools

import jax
import jax.numpy as jnp
import numpy as np
from jax import lax
from jax.experimental import pallas as pl
from jax.experimental.pallas import tpu as pltpu
from jax.experimental.pallas import tpu_sc as plsc

N = 4096
K = 128
R2 = 64.0
NW = 32
TPW = N // NW
CAP = N + 32
BIG = np.float32(3.0e38)
BIGI = np.int32(2**30)


def _sc_radius_graph(posx, posy, posz, batch, bounds):
    mesh = plsc.VectorSubcoreMesh(core_axis_name="c", subcore_axis_name="s")

    @functools.partial(
        pl.kernel,
        out_type=(
            jax.ShapeDtypeStruct((2 * N * K,), jnp.int32),
            jax.ShapeDtypeStruct((N * K,), jnp.float32),
        ),
        mesh=mesh,
        compiler_params=pltpu.CompilerParams(needs_layout_passes=False),
        scratch_types=(
            pltpu.VMEM((N,), jnp.float32),
            pltpu.VMEM((N,), jnp.float32),
            pltpu.VMEM((N,), jnp.float32),
            pltpu.VMEM((TPW,), jnp.int32),
            pltpu.VMEM((16,), jnp.int32),
            pltpu.VMEM((TPW,), jnp.int32),
            pltpu.VMEM((TPW,), jnp.int32),
            pltpu.VMEM((CAP,), jnp.float32),
            pltpu.VMEM((CAP,), jnp.int32),
            pltpu.VMEM((CAP,), jnp.float32),
            pltpu.VMEM((CAP,), jnp.int32),
            pltpu.VMEM((CAP,), jnp.float32),
            pltpu.VMEM((CAP,), jnp.int32),
            pltpu.VMEM((CAP,), jnp.float32),
            pltpu.VMEM((CAP,), jnp.int32),
            pltpu.VMEM((TPW * K,), jnp.int32),
            pltpu.VMEM((TPW * K,), jnp.int32),
            pltpu.VMEM((TPW * K,), jnp.float32),
        ),
    )
    def body(posx_h, posy_h, posz_h, batch_h, bounds_h, out_ei, out_dist,
             px, py, pz, bt, bv, ss, se, cd2, cidx, cd2b, cidxb, cd2c,
             cidxc, cd2d, cidxd, src_loc, tgt_loc, dist_loc):
        wid = lax.axis_index("s") * 2 + lax.axis_index("c")
        base = wid * TPW
        pltpu.sync_copy(posx_h, px)
        pltpu.sync_copy(posy_h, py)
        pltpu.sync_copy(posz_h, pz)
        pltpu.sync_copy(batch_h.at[pl.ds(base, TPW)], bt)
        pltpu.sync_copy(bounds_h, bv)

        neg1 = jnp.full((16,), -1, jnp.int32)
        zeros = jnp.zeros((16,), jnp.float32)
        one = jnp.full((16,), 1, jnp.int32)

        def bounds_body(k, carry):
            b = bt[pl.ds(k * 16, 16)]
            ss[pl.ds(k * 16, 16)] = plsc.load_gather(bv, [b])
            se[pl.ds(k * 16, 16)] = plsc.load_gather(bv, [b + one])
            return carry

        lax.fori_loop(0, TPW // 16, bounds_body, 0)

        def init_body(k, carry):
            src_loc[pl.ds(k * 16, 16)] = neg1
            tgt_loc[pl.ds(k * 16, 16)] = neg1
            dist_loc[pl.ds(k * 16, 16)] = zeros
            return carry

        lax.fori_loop(0, TPW * K // 16, init_body, 0)

        iota = lax.iota(jnp.int32, 16)
        lane0 = iota == 0

        def splat(x, dtype=jnp.int32):
            return jnp.full((16,), x, dtype)

        def post_process(cnt, bd2, bidx, t, i_s):
            bd2[pl.ds(cnt, 16)] = jnp.full((16,), BIG, jnp.float32)
            row = t * K

            def slow_select():
                nsel = jnp.minimum(cnt, K)
                nch = (cnt + 15) >> 4

                def sel_body(sl, carry):
                    def minpass(c, macc):
                        return jnp.minimum(macc, bd2[pl.ds(c * 16, 16)])

                    macc = lax.fori_loop(0, nch, minpass,
                                         jnp.full((16,), BIG, jnp.float32))
                    m = jnp.min(macc)

                    def idxpass(c, jacc):
                        d2c = bd2[pl.ds(c * 16, 16)]
                        ic = bidx[pl.ds(c * 16, 16)]
                        return jnp.minimum(jacc, jnp.where(d2c == m, ic, BIGI))

                    jacc = lax.fori_loop(0, nch, idxpass,
                                         jnp.full((16,), BIGI, jnp.int32))
                    jmin = jnp.min(jacc)
                    p_s = splat(row + sl)
                    plsc.store_scatter(src_loc, [p_s], splat(jmin), mask=lane0)
                    plsc.store_scatter(dist_loc, [p_s], splat(m, jnp.float32),
                                       mask=lane0)

                    def knock(c, kc):
                        d2c = bd2[pl.ds(c * 16, 16)]
                        ic = bidx[pl.ds(c * 16, 16)]
                        hit = (d2c == m) & (ic == jmin)
                        bd2[pl.ds(c * 16, 16)] = jnp.where(hit, BIG, d2c)
                        return kc

                    lax.fori_loop(0, nch, knock, 0)
                    return carry

                lax.fori_loop(0, nsel, sel_body, 0)

            def tie_fallback(tie):
                anytie = plsc.all_reduce_population_count(tie)[0] > 0

                @pl.when(anytie)
                def _():
                    slow_select()

            @pl.when(cnt <= 16)
            def _():
                sk, sv = plsc.sort_key_val(bd2[pl.ds(0, 16)],
                                           bidx[pl.ds(0, 16)])
                pref = iota < cnt
                plsc.store_compressed(src_loc.at[pl.ds(row, 16)], sv,
                                      mask=pref)
                plsc.store_compressed(dist_loc.at[pl.ds(row, 16)], sk,
                                      mask=pref)
                a = dist_loc[pl.ds(row, 16)]
                b = dist_loc[pl.ds(row + 1, 16)]
                tie_fallback((a == b) & (iota < cnt - 1))

            @pl.when((cnt > 16) & (cnt <= 32))
            def _():
                ad, ai = plsc.sort_key_val(bd2[pl.ds(0, 16)],
                                           bidx[pl.ds(0, 16)])
                bd, bi = plsc.sort_key_val(bd2[pl.ds(16, 16)],
                                           bidx[pl.ds(16, 16)])
                brd = jnp.flip(bd)
                bri = jnp.flip(bi)
                sel = (ad < brd) | ((ad == brd) & (ai < bri))
                ld = jnp.where(sel, ad, brd)
                li = jnp.where(sel, ai, bri)
                hd = jnp.where(sel, brd, ad)
                hi = jnp.where(sel, bri, ai)
                ld, li = plsc.sort_key_val(ld, li)
                hd, hi = plsc.sort_key_val(hd, hi)
                src_loc[pl.ds(row, 16)] = li
                dist_loc[pl.ds(row, 16)] = ld
                pref = iota < cnt - 16
                plsc.store_compressed(src_loc.at[pl.ds(row + 16, 16)], hi,
                                      mask=pref)
                plsc.store_compressed(dist_loc.at[pl.ds(row + 16, 16)], hd,
                                      mask=pref)
                a1 = dist_loc[pl.ds(row, 16)]
                b1 = dist_loc[pl.ds(row + 1, 16)]
                a2 = dist_loc[pl.ds(row + 16, 16)]
                b2 = dist_loc[pl.ds(row + 17, 16)]
                tie1 = (a1 == b1) & (iota < 15)
                tie2 = (a2 == b2) & (iota < cnt - 17)
                tie_fallback(tie1 | tie2)

            @pl.when(cnt > 32)
            def _():
                slow_select()

            def sqrt_body(k, carry2):
                x = dist_loc[pl.ds(row + k * 16, 16)]
                sv = src_loc[pl.ds(row + k * 16, 16)]
                xm = jnp.maximum(x, jnp.float32(1e-12))
                ib = plsc.bitcast(xm, jnp.int32)
                r = plsc.bitcast(jnp.int32(0x5F3759DF) - (ib >> 1),
                                 jnp.float32)
                hx = xm * jnp.float32(0.5)
                r = r * (jnp.float32(1.5) - hx * r * r)
                r = r * (jnp.float32(1.5) - hx * r * r)
                r = r * (jnp.float32(1.5) - hx * r * r)
                d = xm * r
                d = jnp.float32(0.5) * (d + xm / d)
                sel = sv >= 0
                dist_loc[pl.ds(row + k * 16, 16)] = jnp.where(
                    sel, d, jnp.float32(0.0))
                tgt_loc[pl.ds(row + k * 16, 16)] = jnp.where(sel, i_s, neg1)
                return carry2

            lax.fori_loop(0, (jnp.minimum(cnt, K) + 15) >> 4, sqrt_body, 0)

        bufs = ((cd2, cidx), (cd2b, cidxb), (cd2c, cidxc), (cd2d, cidxd))
        NT = len(bufs)

        def group_body(tg, carry):
            ts = [NT * tg + u for u in range(NT)]
            i_ss = [splat(base + t) for t in ts]
            xs = [plsc.load_gather(px, [i_s]) for i_s in i_ss]
            ys = [plsc.load_gather(py, [i_s]) for i_s in i_ss]
            zs = [plsc.load_gather(pz, [i_s]) for i_s in i_ss]
            s0s = [plsc.load_gather(ss, [splat(t)])[0] for t in ts]
            s1s = [plsc.load_gather(se, [splat(t)])[0] for t in ts]
            lo = s0s[0]
            hi = s1s[0]
            for u in range(1, NT):
                lo = jnp.minimum(lo, s0s[u])
                hi = jnp.maximum(hi, s1s[u])
            c0 = lo & jnp.int32(-16)
            nsteps = (hi - c0 + 15) >> 4

            def make_chunk_body(shared_bounds):
                def chunk_body(c, cnts):
                    cb = c0 + c * 16
                    jv = cb + iota
                    pxc = px[pl.ds(cb, 16)]
                    pyc = py[pl.ds(cb, 16)]
                    pzc = pz[pl.ds(cb, 16)]
                    if shared_bounds:
                        sm = (jv >= lo) & (jv < hi)
                    out = []
                    for u in range(NT):
                        dx = pxc - xs[u]
                        dy = pyc - ys[u]
                        dz = pzc - zs[u]
                        d2 = (dx * dx + dy * dy) + dz * dz
                        v = (d2 <= R2) & (jv != base + ts[u])
                        if shared_bounds:
                            v &= sm
                        else:
                            v &= (jv >= s0s[u]) & (jv < s1s[u])
                        nv = plsc.all_reduce_population_count(v)[0]
                        bd2, bidx = bufs[u]
                        plsc.store_compressed(bd2.at[pl.ds(cnts[u], 16)], d2,
                                              mask=v)
                        plsc.store_compressed(bidx.at[pl.ds(cnts[u], 16)], jv,
                                              mask=v)
                        out.append(cnts[u] + nv)
                    return tuple(out)

                return chunk_body

            zero_cnts = tuple(jnp.int32(0) for _ in range(NT))
            cnts = lax.fori_loop(0, nsteps, make_chunk_body(False), zero_cnts)
            for u in range(NT):
                post_process(cnts[u], bufs[u][0], bufs[u][1], ts[u], i_ss[u])
            return carry

        lax.fori_loop(0, TPW // NT, group_body, 0)

        pltpu.sync_copy(src_loc, out_ei.at[pl.ds(base * K, TPW * K)])
        pltpu.sync_copy(tgt_loc, out_ei.at[pl.ds(N * K + base * K, TPW * K)])
        pltpu.sync_copy(dist_loc, out_dist.at[pl.ds(base * K, TPW * K)])

    return body(posx, posy, posz, batch, bounds)


def kernel(pos, batch):
    batch = batch.astype(jnp.int32)
    posx = pos[:, 0]
    posy = pos[:, 1]
    posz = pos[:, 2]
    bounds = jnp.sum(
        (batch[None, :] < jnp.arange(9, dtype=jnp.int32)[:, None]).astype(jnp.int32),
        axis=1)
    bounds16 = jnp.pad(bounds, (0, 7))
    ei, dist = _sc_radius_graph(posx, posy, posz, batch, bounds16)
    return ei.reshape(2, N * K), dist.reshape(N, K)


# speedup vs baseline: 1.4053x; 1.0165x over previous
import functools

import jax
import jax.numpy as jnp
import numpy as np
from jax import lax
from jax.experimental import pallas as pl
from jax.experimental.pallas import tpu as pltpu
from jax.experimental.pallas import tpu_sc as plsc

N = 4096
K = 128
R2 = 64.0
NW = 32
TPW = N // NW
CAP = N + 32
BIG = np.float32(3.0e38)
BIGI = np.int32(2**30)


def _sc_radius_graph(posx, posy, posz, batch, bounds):
    mesh = plsc.VectorSubcoreMesh(core_axis_name="c", subcore_axis_name="s")

    @functools.partial(
        pl.kernel,
        out_type=(
            jax.ShapeDtypeStruct((2 * N * K,), jnp.int32),
            jax.ShapeDtypeStruct((N * K,), jnp.float32),
        ),
        mesh=mesh,
        compiler_params=pltpu.CompilerParams(needs_layout_passes=False),
        scratch_types=(
            pltpu.VMEM((N,), jnp.float32),
            pltpu.VMEM((N,), jnp.float32),
            pltpu.VMEM((N,), jnp.float32),
            pltpu.VMEM((TPW,), jnp.int32),
            pltpu.VMEM((16,), jnp.int32),
            pltpu.VMEM((TPW,), jnp.int32),
            pltpu.VMEM((TPW,), jnp.int32),
            pltpu.VMEM((CAP,), jnp.float32),
            pltpu.VMEM((CAP,), jnp.int32),
            pltpu.VMEM((CAP,), jnp.float32),
            pltpu.VMEM((CAP,), jnp.int32),
            pltpu.VMEM((CAP,), jnp.float32),
            pltpu.VMEM((CAP,), jnp.int32),
            pltpu.VMEM((CAP,), jnp.float32),
            pltpu.VMEM((CAP,), jnp.int32),
            pltpu.VMEM((TPW * K,), jnp.int32),
            pltpu.VMEM((TPW * K,), jnp.int32),
            pltpu.VMEM((TPW * K,), jnp.float32),
        ),
    )
    def body(posx_h, posy_h, posz_h, batch_h, bounds_h, out_ei, out_dist,
             px, py, pz, bt, bv, ss, se, cd2, cidx, cd2b, cidxb, cd2c,
             cidxc, cd2d, cidxd, src_loc, tgt_loc, dist_loc):
        wid = lax.axis_index("s") * 2 + lax.axis_index("c")
        base = wid * TPW
        pltpu.sync_copy(posx_h, px)
        pltpu.sync_copy(posy_h, py)
        pltpu.sync_copy(posz_h, pz)
        pltpu.sync_copy(batch_h.at[pl.ds(base, TPW)], bt)
        pltpu.sync_copy(bounds_h, bv)

        neg1 = jnp.full((16,), -1, jnp.int32)
        zeros = jnp.zeros((16,), jnp.float32)
        one = jnp.full((16,), 1, jnp.int32)

        def bounds_body(k, carry):
            b = bt[pl.ds(k * 16, 16)]
            ss[pl.ds(k * 16, 16)] = plsc.load_gather(bv, [b])
            se[pl.ds(k * 16, 16)] = plsc.load_gather(bv, [b + one])
            return carry

        lax.fori_loop(0, TPW // 16, bounds_body, 0)

        def init_body(k, carry):
            src_loc[pl.ds(k * 16, 16)] = neg1
            tgt_loc[pl.ds(k * 16, 16)] = neg1
            dist_loc[pl.ds(k * 16, 16)] = zeros
            return carry

        lax.fori_loop(0, TPW * K // 16, init_body, 0)

        iota = lax.iota(jnp.int32, 16)
        lane0 = iota == 0

        def splat(x, dtype=jnp.int32):
            return jnp.full((16,), x, dtype)

        def post_process(cnt, bd2, bidx, t, i_s):
            bd2[pl.ds(cnt, 16)] = jnp.full((16,), BIG, jnp.float32)
            row = t * K

            def slow_select():
                nsel = jnp.minimum(cnt, K)
                nch = (cnt + 15) >> 4

                def sel_body(sl, carry):
                    def minpass(c, macc):
                        return jnp.minimum(macc, bd2[pl.ds(c * 16, 16)])

                    macc = lax.fori_loop(0, nch, minpass,
                                         jnp.full((16,), BIG, jnp.float32))
                    m = jnp.min(macc)

                    def idxpass(c, jacc):
                        d2c = bd2[pl.ds(c * 16, 16)]
                        ic = bidx[pl.ds(c * 16, 16)]
                        return jnp.minimum(jacc, jnp.where(d2c == m, ic, BIGI))

                    jacc = lax.fori_loop(0, nch, idxpass,
                                         jnp.full((16,), BIGI, jnp.int32))
                    jmin = jnp.min(jacc)
                    p_s = splat(row + sl)
                    plsc.store_scatter(src_loc, [p_s], splat(jmin), mask=lane0)
                    plsc.store_scatter(dist_loc, [p_s], splat(m, jnp.float32),
                                       mask=lane0)

                    def knock(c, kc):
                        d2c = bd2[pl.ds(c * 16, 16)]
                        ic = bidx[pl.ds(c * 16, 16)]
                        hit = (d2c == m) & (ic == jmin)
                        bd2[pl.ds(c * 16, 16)] = jnp.where(hit, BIG, d2c)
                        return kc

                    lax.fori_loop(0, nch, knock, 0)
                    return carry

                lax.fori_loop(0, nsel, sel_body, 0)

            def tie_fallback(tie):
                anytie = plsc.all_reduce_population_count(tie)[0] > 0

                @pl.when(anytie)
                def _():
                    slow_select()

            @pl.when(cnt <= 16)
            def _():
                sk, sv = plsc.sort_key_val(bd2[pl.ds(0, 16)],
                                           bidx[pl.ds(0, 16)])
                pref = iota < cnt
                plsc.store_compressed(src_loc.at[pl.ds(row, 16)], sv,
                                      mask=pref)
                plsc.store_compressed(dist_loc.at[pl.ds(row, 16)], sk,
                                      mask=pref)
                a = dist_loc[pl.ds(row, 16)]
                b = dist_loc[pl.ds(row + 1, 16)]
                tie_fallback((a == b) & (iota < cnt - 1))

            @pl.when((cnt > 16) & (cnt <= 32))
            def _():
                ad, ai = plsc.sort_key_val(bd2[pl.ds(0, 16)],
                                           bidx[pl.ds(0, 16)])
                bd, bi = plsc.sort_key_val(bd2[pl.ds(16, 16)],
                                           bidx[pl.ds(16, 16)])
                brd = jnp.flip(bd)
                bri = jnp.flip(bi)
                sel = (ad < brd) | ((ad == brd) & (ai < bri))
                ld = jnp.where(sel, ad, brd)
                li = jnp.where(sel, ai, bri)
                hd = jnp.where(sel, brd, ad)
                hi = jnp.where(sel, bri, ai)
                ld, li = plsc.sort_key_val(ld, li)
                hd, hi = plsc.sort_key_val(hd, hi)
                src_loc[pl.ds(row, 16)] = li
                dist_loc[pl.ds(row, 16)] = ld
                pref = iota < cnt - 16
                plsc.store_compressed(src_loc.at[pl.ds(row + 16, 16)], hi,
                                      mask=pref)
                plsc.store_compressed(dist_loc.at[pl.ds(row + 16, 16)], hd,
                                      mask=pref)
                a1 = dist_loc[pl.ds(row, 16)]
                b1 = dist_loc[pl.ds(row + 1, 16)]
                a2 = dist_loc[pl.ds(row + 16, 16)]
                b2 = dist_loc[pl.ds(row + 17, 16)]
                tie1 = (a1 == b1) & (iota < 15)
                tie2 = (a2 == b2) & (iota < cnt - 17)
                tie_fallback(tie1 | tie2)

            @pl.when(cnt > 32)
            def _():
                slow_select()

            def sqrt_body(k, carry2):
                x = dist_loc[pl.ds(row + k * 16, 16)]
                sv = src_loc[pl.ds(row + k * 16, 16)]
                xm = jnp.maximum(x, jnp.float32(1e-12))
                ib = plsc.bitcast(xm, jnp.int32)
                r = plsc.bitcast(jnp.int32(0x5F3759DF) - (ib >> 1),
                                 jnp.float32)
                hx = xm * jnp.float32(0.5)
                r = r * (jnp.float32(1.5) - hx * r * r)
                r = r * (jnp.float32(1.5) - hx * r * r)
                r = r * (jnp.float32(1.5) - hx * r * r)
                d = xm * r
                sel = sv >= 0
                dist_loc[pl.ds(row + k * 16, 16)] = jnp.where(
                    sel, d, jnp.float32(0.0))
                tgt_loc[pl.ds(row + k * 16, 16)] = jnp.where(sel, i_s, neg1)
                return carry2

            lax.fori_loop(0, (jnp.minimum(cnt, K) + 15) >> 4, sqrt_body, 0)

        bufs = ((cd2, cidx), (cd2b, cidxb), (cd2c, cidxc), (cd2d, cidxd))
        NT = len(bufs)

        def group_body(tg, carry):
            ts = [NT * tg + u for u in range(NT)]
            i_ss = [splat(base + t) for t in ts]
            xs = [plsc.load_gather(px, [i_s]) for i_s in i_ss]
            ys = [plsc.load_gather(py, [i_s]) for i_s in i_ss]
            zs = [plsc.load_gather(pz, [i_s]) for i_s in i_ss]
            s0s = [plsc.load_gather(ss, [splat(t)])[0] for t in ts]
            s1s = [plsc.load_gather(se, [splat(t)])[0] for t in ts]
            lo = s0s[0]
            hi = s1s[0]
            for u in range(1, NT):
                lo = jnp.minimum(lo, s0s[u])
                hi = jnp.maximum(hi, s1s[u])
            c0 = lo & jnp.int32(-16)
            nsteps = (hi - c0 + 15) >> 4

            def make_chunk_body(shared_bounds):
                def chunk_body(c, cnts):
                    cb = c0 + c * 16
                    jv = cb + iota
                    pxc = px[pl.ds(cb, 16)]
                    pyc = py[pl.ds(cb, 16)]
                    pzc = pz[pl.ds(cb, 16)]
                    if shared_bounds:
                        sm = (jv >= lo) & (jv < hi)
                    out = []
                    for u in range(NT):
                        dx = pxc - xs[u]
                        dy = pyc - ys[u]
                        dz = pzc - zs[u]
                        d2 = (dx * dx + dy * dy) + dz * dz
                        v = (d2 <= R2) & (jv != base + ts[u])
                        if shared_bounds:
                            v &= sm
                        else:
                            v &= (jv >= s0s[u]) & (jv < s1s[u])
                        nv = plsc.all_reduce_population_count(v)[0]
                        bd2, bidx = bufs[u]
                        plsc.store_compressed(bd2.at[pl.ds(cnts[u], 16)], d2,
                                              mask=v)
                        plsc.store_compressed(bidx.at[pl.ds(cnts[u], 16)], jv,
                                              mask=v)
                        out.append(cnts[u] + nv)
                    return tuple(out)

                return chunk_body

            zero_cnts = tuple(jnp.int32(0) for _ in range(NT))
            cnts = lax.fori_loop(0, nsteps, make_chunk_body(False), zero_cnts)
            for u in range(NT):
                post_process(cnts[u], bufs[u][0], bufs[u][1], ts[u], i_ss[u])
            return carry

        lax.fori_loop(0, TPW // NT, group_body, 0)

        pltpu.sync_copy(src_loc, out_ei.at[pl.ds(base * K, TPW * K)])
        pltpu.sync_copy(tgt_loc, out_ei.at[pl.ds(N * K + base * K, TPW * K)])
        pltpu.sync_copy(dist_loc, out_dist.at[pl.ds(base * K, TPW * K)])

    return body(posx, posy, posz, batch, bounds)


def kernel(pos, batch):
    batch = batch.astype(jnp.int32)
    posx = pos[:, 0]
    posy = pos[:, 1]
    posz = pos[:, 2]
    bounds = jnp.sum(
        (batch[None, :] < jnp.arange(9, dtype=jnp.int32)[:, None]).astype(jnp.int32),
        axis=1)
    bounds16 = jnp.pad(bounds, (0, 7))
    ei, dist = _sc_radius_graph(posx, posy, posz, batch, bounds16)
    return ei.reshape(2, N * K), dist.reshape(N, K)
